# baseline (speedup 1.0000x reference)
import functools

import jax
import jax.numpy as jnp
import numpy as np
from jax.experimental import pallas as pl
from jax.experimental.pallas import tpu as pltpu

f32 = jnp.float32
bf16 = jnp.bfloat16

_BBA = 2048
_BBS = 8192
_BBC = 2048


def _tlhs(a_t, w):
    return jax.lax.dot_general(a_t, w, (((0,), (0,)), ((), ())),
                               preferred_element_type=f32)


def _enc_kernel(sc_s, xs_r, eps_r, ew1_r, eb1_r, ew2_r, eb2_r, qzw_r, qzb_r,
                c0_r, c1_r, z0_r, okl_r, *, n):
    ew1c = jnp.swapaxes(ew1_r[...], 0, 1)
    eb1c = jnp.swapaxes(eb1_r[...], 0, 1)
    ew2 = ew2_r[...]
    eb2 = eb2_r[...]
    pzm = sc_s[0]
    pzls = sc_s[1]

    ctx0 = None
    for t in range(n):
        x = xs_r[t, 0:1, :]
        h1t = jnp.maximum(ew1c * x + eb1c, 0.0)
        ctx = _tlhs(h1t, ew2) + eb2
        ct_rows = jnp.swapaxes(ctx, 0, 1).astype(bf16)
        c0_r[t, 0:1, :] = ct_rows[0:1, :]
        c1_r[t, 0:1, :] = ct_rows[1:2, :]
        if t == 0:
            ctx0 = ctx.astype(bf16)

    q = jnp.dot(ctx0, qzw_r[...], preferred_element_type=f32) + qzb_r[...]
    q_rows = jnp.swapaxes(q, 0, 1)
    qm, qls = q_rows[0:1, :], q_rows[1:2, :]
    z0 = qm + jnp.exp(qls) * eps_r[...]
    kl = (pzls - qls) + (jnp.exp(2.0 * qls) + (qm - pzm) ** 2) / (
        2.0 * jnp.exp(2.0 * pzls)) - 0.5
    z0_r[...] = z0
    okl_r[0, 0, 0] = jnp.sum(kl)


def _scan_kernel(ts_s, sq_s, sc_s, c0_r, c1_r, bm_r, z0_r,
                 fw1t_r, fb1_r, fw2t_r, ozs_r, oqp_r, *, n_steps):
    kap = sc_s[2]
    th = sc_s[3]
    sig = sc_s[4]
    fb2 = sc_s[5]
    fw1t = fw1t_r[...]
    fb1 = fb1_r[...]
    fw2t = fw2t_r[...]

    z0 = z0_r[...]
    ozs_r[0, 0:1, :] = z0

    def body(t, carry):
        z, acc_q = carry
        zb = z.astype(bf16)
        c0 = c0_r[pl.ds(t + 1, 1), 0, :]
        c1 = c1_r[pl.ds(t + 1, 1), 0, :]
        in3 = jnp.concatenate([zb, c0, c1], axis=0)
        pre = jnp.dot(fw1t, in3, preferred_element_type=f32) + fb1
        h = jnp.maximum(pre, 0.0).astype(bf16)
        fr = jnp.dot(fw2t, h, preferred_element_type=f32) + fb2
        dt = ts_s[t + 1] - ts_s[t]
        u = (fr - kap * (th - z)) / sig
        acc_q = acc_q + (u * u) * (0.5 * dt)
        dw = bm_r[pl.ds(t, 1), 0, :] * sq_s[t]
        z = (z + fr * dt) + sig * dw
        ozs_r[pl.ds(t + 1, 1), 0, :] = z
        return z, acc_q

    _, acc_q = jax.lax.fori_loop(0, n_steps, body,
                                 (z0, jnp.zeros_like(z0)))
    oqp_r[0, 0, 0] = jnp.sum(acc_q)


def _proj_kernel(sc_s, zs_r, xs_r, pw1_r, pb1_r, pw2_r, pb2_r, pw3_r,
                 oxh_r, olp_r, *, bb):
    pb3 = sc_s[6]
    std = sc_s[7]
    pw1 = pw1_r[...]
    pb1 = pb1_r[...]

    z_row = zs_r[0, 0:1, :]
    z = _tlhs(z_row, jnp.ones((1, 1), f32))
    a = z * pw1 + pb1
    p1 = jnp.maximum(a, 0.0) + jnp.log1p(jnp.exp(-jnp.abs(a)))
    p2 = jnp.dot(p1, pw2_r[...], preferred_element_type=f32) + pb2_r[...]
    p2 = jnp.maximum(p2, 0.0).astype(bf16)
    xh = jnp.dot(p2, pw3_r[...], preferred_element_type=f32) + pb3
    xh_row = jnp.swapaxes(xh, 0, 1)
    oxh_r[0, 0:1, :] = xh_row
    r = (xs_r[0, 0:1, :] - xh_row) / std
    log2pi = np.float32(np.log(2.0 * np.pi))
    olp_r[0, 0, 0] = (-0.5 * jnp.sum(r * r)
                      - np.float32(bb) * (jnp.log(std) + 0.5 * log2pi))


def kernel(xs, ts, noise_std, eps0, bm_eps, enc_w1, enc_b1, enc_w2, enc_b2,
           qz0_w, qz0_b, f_w1, f_b1, f_w2, f_b2,
           proj_w1, proj_b1, proj_w2, proj_b2, proj_w3, proj_b3,
           pz0_mean, pz0_logstd, kappa, theta, sigma):
    n, b, _ = xs.shape
    hid = enc_w1.shape[1]

    sc = jnp.concatenate([
        pz0_mean.reshape(-1), pz0_logstd.reshape(-1),
        kappa.reshape(-1), theta.reshape(-1), sigma.reshape(-1),
        f_b2.reshape(-1), proj_b3.reshape(-1),
        jnp.asarray(noise_std, f32).reshape(-1),
    ]).astype(f32)
    sqdts = jnp.sqrt(ts[1:] - ts[:-1])

    xs_tm = xs.reshape(n, 1, b)
    bm_tm = bm_eps.reshape(n - 1, 1, b)
    eps_row = eps0.reshape(1, b)

    smem = functools.partial(pl.BlockSpec, memory_space=pltpu.SMEM)
    full = lambda shape: pl.BlockSpec(shape, lambda *i: (0,) * len(shape))

    ga = b // _BBA
    c0_tm, c1_tm, z0_row, kl_p = pl.pallas_call(
        functools.partial(_enc_kernel, n=n),
        grid=(ga,),
        in_specs=[
            smem(),
            pl.BlockSpec((n, 1, _BBA), lambda i: (0, 0, i)),
            pl.BlockSpec((1, _BBA), lambda i: (0, i)),
            full((1, hid)), full((1, hid)),
            full((hid, 2)), full((1, 2)),
            full((2, 2)), full((1, 2)),
        ],
        out_specs=[
            pl.BlockSpec((n, 1, _BBA), lambda i: (0, 0, i)),
            pl.BlockSpec((n, 1, _BBA), lambda i: (0, 0, i)),
            pl.BlockSpec((1, _BBA), lambda i: (0, i)),
            smem((1, 1, 1), lambda i: (i, 0, 0)),
        ],
        out_shape=[
            jax.ShapeDtypeStruct((n, 1, b), bf16),
            jax.ShapeDtypeStruct((n, 1, b), bf16),
            jax.ShapeDtypeStruct((1, b), f32),
            jax.ShapeDtypeStruct((ga, 1, 1), f32),
        ],
        compiler_params=pltpu.CompilerParams(
            dimension_semantics=("parallel",),
            vmem_limit_bytes=100 * 1024 * 1024,
        ),
    )(sc, xs_tm, eps_row, enc_w1, enc_b1.reshape(1, hid), enc_w2,
      enc_b2.reshape(1, 2), qz0_w, qz0_b.reshape(1, 2))

    gs = b // _BBS
    fw1t = f_w1.astype(bf16).T
    fw2t = f_w2.astype(bf16).T
    zs_tm, qp_p = pl.pallas_call(
        functools.partial(_scan_kernel, n_steps=n - 1),
        grid=(gs,),
        in_specs=[
            smem(), smem(), smem(),
            pl.BlockSpec((n, 1, _BBS), lambda i: (0, 0, i)),
            pl.BlockSpec((n, 1, _BBS), lambda i: (0, 0, i)),
            pl.BlockSpec((n - 1, 1, _BBS), lambda i: (0, 0, i)),
            pl.BlockSpec((1, _BBS), lambda i: (0, i)),
            full((hid, 3)), full((hid, 1)), full((1, hid)),
        ],
        out_specs=[
            pl.BlockSpec((n, 1, _BBS), lambda i: (0, 0, i)),
            smem((1, 1, 1), lambda i: (i, 0, 0)),
        ],
        out_shape=[
            jax.ShapeDtypeStruct((n, 1, b), f32),
            jax.ShapeDtypeStruct((gs, 1, 1), f32),
        ],
        compiler_params=pltpu.CompilerParams(
            dimension_semantics=("parallel",),
            vmem_limit_bytes=100 * 1024 * 1024,
        ),
    )(ts, sqdts, sc, c0_tm, c1_tm, bm_tm, z0_row,
      fw1t, f_b1.reshape(hid, 1), fw2t)

    gc = b // _BBC
    xh_tm, lp_p = pl.pallas_call(
        functools.partial(_proj_kernel, bb=_BBC),
        grid=(n, gc),
        in_specs=[
            smem(),
            pl.BlockSpec((1, 1, _BBC), lambda t, i: (t, 0, i)),
            pl.BlockSpec((1, 1, _BBC), lambda t, i: (t, 0, i)),
            full((1, hid)), full((1, hid)),
            full((hid, hid)), full((1, hid)),
            full((hid, 1)),
        ],
        out_specs=[
            pl.BlockSpec((1, 1, _BBC), lambda t, i: (t, 0, i)),
            smem((1, 1, 1), lambda t, i: (t * gc + i, 0, 0)),
        ],
        out_shape=[
            jax.ShapeDtypeStruct((n, 1, b), f32),
            jax.ShapeDtypeStruct((n * gc, 1, 1), f32),
        ],
        compiler_params=pltpu.CompilerParams(
            dimension_semantics=("parallel", "parallel"),
            vmem_limit_bytes=100 * 1024 * 1024,
        ),
    )(sc, zs_tm, xs_tm, proj_w1, proj_b1.reshape(1, hid),
      proj_w2, proj_b2.reshape(1, hid), proj_w3)

    inv_b = 1.0 / np.float32(b)
    log_pxs = jnp.sum(lp_p) * inv_b
    logqp = (jnp.sum(kl_p) + jnp.sum(qp_p)) * inv_b
    return log_pxs, logqp, xh_tm.reshape(n, b, 1)


# speedup vs baseline: 1.2689x; 1.2689x over previous
import functools

import jax
import jax.numpy as jnp
import numpy as np
from jax.experimental import pallas as pl
from jax.experimental.pallas import tpu as pltpu

f32 = jnp.float32
bf16 = jnp.bfloat16

_BBA = 2048
_BBS = 8192
_BBC = 2048


def _tlhs(a_t, w):
    return jax.lax.dot_general(a_t, w, (((0,), (0,)), ((), ())),
                               preferred_element_type=f32)


def _enc_kernel(sc_s, xs_r, eps_r, ew1_r, eb1_r, ew2_r, eb2_r, qzw_r, qzb_r,
                c0_r, c1_r, z0_r, okl_r, *, n):
    ew1c = jnp.swapaxes(ew1_r[...], 0, 1)
    eb1c = jnp.swapaxes(eb1_r[...], 0, 1)
    ew2 = ew2_r[...]
    eb2 = eb2_r[...]
    pzm = sc_s[0]
    pzls = sc_s[1]

    ctx0 = None
    for t in range(n):
        x = xs_r[t, 0:1, :]
        h1t = jnp.maximum(ew1c * x + eb1c, 0.0)
        ctx = _tlhs(h1t, ew2) + eb2
        ct_rows = jnp.swapaxes(ctx, 0, 1).astype(bf16)
        c0_r[t, 0:1, :] = ct_rows[0:1, :]
        c1_r[t, 0:1, :] = ct_rows[1:2, :]
        if t == 0:
            ctx0 = ctx.astype(bf16)

    q = jnp.dot(ctx0, qzw_r[...], preferred_element_type=f32) + qzb_r[...]
    q_rows = jnp.swapaxes(q, 0, 1)
    qm, qls = q_rows[0:1, :], q_rows[1:2, :]
    z0 = qm + jnp.exp(qls) * eps_r[...]
    kl = (pzls - qls) + (jnp.exp(2.0 * qls) + (qm - pzm) ** 2) / (
        2.0 * jnp.exp(2.0 * pzls)) - 0.5
    z0_r[...] = z0
    okl_r[0, 0, 0] = jnp.sum(kl)


def _scan_kernel(ts_s, sq_s, sc_s, c0_r, c1_r, bm_r, z0_r,
                 fw1t_r, fb1_r, fw2t_r, ozs_r, oqp_r, *, n_steps):
    kap = sc_s[2]
    th = sc_s[3]
    sig = sc_s[4]
    fb2 = sc_s[5]
    fw1t = fw1t_r[...]
    fb1 = fb1_r[...]
    fw2t = fw2t_r[...]

    z0 = z0_r[...]
    ozs_r[0, 0:1, :] = z0

    def body(t, carry):
        z, acc_q = carry
        zb = z.astype(bf16)
        c0 = c0_r[pl.ds(t + 1, 1), 0, :]
        c1 = c1_r[pl.ds(t + 1, 1), 0, :]
        in3 = jnp.concatenate([zb, c0, c1], axis=0)
        pre = jnp.dot(fw1t, in3, preferred_element_type=f32) + fb1
        h = jnp.maximum(pre, 0.0).astype(bf16)
        fr = jnp.dot(fw2t, h, preferred_element_type=f32) + fb2
        dt = ts_s[t + 1] - ts_s[t]
        u = (fr - kap * (th - z)) / sig
        acc_q = acc_q + (u * u) * (0.5 * dt)
        dw = bm_r[pl.ds(t, 1), 0, :] * sq_s[t]
        z = (z + fr * dt) + sig * dw
        ozs_r[pl.ds(t + 1, 1), 0, :] = z
        return z, acc_q

    _, acc_q = jax.lax.fori_loop(0, n_steps, body,
                                 (z0, jnp.zeros_like(z0)))
    oqp_r[0, 0, 0] = jnp.sum(acc_q)


def _proj_kernel(sc_s, zs_r, xs_r, pw1_r, pb1_r, pw2_r, pb2_r, pw3_r,
                 oxh_r, olp_r, *, bb):
    pb3 = sc_s[6]
    std = sc_s[7]
    pw1c = jnp.swapaxes(pw1_r[...], 0, 1)
    pb1c = jnp.swapaxes(pb1_r[...], 0, 1)

    z = zs_r[0, 0:1, :]
    a = pw1c * z + pb1c
    p1t = jnp.maximum(a, 0.0) + jnp.log1p(jnp.exp(-jnp.abs(a)))
    p2 = _tlhs(p1t, pw2_r[...]) + pb2_r[...]
    p2 = jnp.maximum(p2, 0.0).astype(bf16)
    xh = jnp.dot(p2, pw3_r[...], preferred_element_type=f32) + pb3
    xh_row = jnp.swapaxes(xh, 0, 1)
    oxh_r[0, 0:1, :] = xh_row
    r = (xs_r[0, 0:1, :] - xh_row) / std
    log2pi = np.float32(np.log(2.0 * np.pi))
    olp_r[0, 0, 0] = (-0.5 * jnp.sum(r * r)
                      - np.float32(bb) * (jnp.log(std) + 0.5 * log2pi))


def kernel(xs, ts, noise_std, eps0, bm_eps, enc_w1, enc_b1, enc_w2, enc_b2,
           qz0_w, qz0_b, f_w1, f_b1, f_w2, f_b2,
           proj_w1, proj_b1, proj_w2, proj_b2, proj_w3, proj_b3,
           pz0_mean, pz0_logstd, kappa, theta, sigma):
    n, b, _ = xs.shape
    hid = enc_w1.shape[1]

    sc = jnp.concatenate([
        pz0_mean.reshape(-1), pz0_logstd.reshape(-1),
        kappa.reshape(-1), theta.reshape(-1), sigma.reshape(-1),
        f_b2.reshape(-1), proj_b3.reshape(-1),
        jnp.asarray(noise_std, f32).reshape(-1),
    ]).astype(f32)
    sqdts = jnp.sqrt(ts[1:] - ts[:-1])

    xs_tm = xs.reshape(n, 1, b)
    bm_tm = bm_eps.reshape(n - 1, 1, b)
    eps_row = eps0.reshape(1, b)

    smem = functools.partial(pl.BlockSpec, memory_space=pltpu.SMEM)
    full = lambda shape: pl.BlockSpec(shape, lambda *i: (0,) * len(shape))

    ga = b // _BBA
    c0_tm, c1_tm, z0_row, kl_p = pl.pallas_call(
        functools.partial(_enc_kernel, n=n),
        grid=(ga,),
        in_specs=[
            smem(),
            pl.BlockSpec((n, 1, _BBA), lambda i: (0, 0, i)),
            pl.BlockSpec((1, _BBA), lambda i: (0, i)),
            full((1, hid)), full((1, hid)),
            full((hid, 2)), full((1, 2)),
            full((2, 2)), full((1, 2)),
        ],
        out_specs=[
            pl.BlockSpec((n, 1, _BBA), lambda i: (0, 0, i)),
            pl.BlockSpec((n, 1, _BBA), lambda i: (0, 0, i)),
            pl.BlockSpec((1, _BBA), lambda i: (0, i)),
            smem((1, 1, 1), lambda i: (i, 0, 0)),
        ],
        out_shape=[
            jax.ShapeDtypeStruct((n, 1, b), bf16),
            jax.ShapeDtypeStruct((n, 1, b), bf16),
            jax.ShapeDtypeStruct((1, b), f32),
            jax.ShapeDtypeStruct((ga, 1, 1), f32),
        ],
        compiler_params=pltpu.CompilerParams(
            dimension_semantics=("parallel",),
            vmem_limit_bytes=100 * 1024 * 1024,
        ),
    )(sc, xs_tm, eps_row, enc_w1, enc_b1.reshape(1, hid), enc_w2,
      enc_b2.reshape(1, 2), qz0_w, qz0_b.reshape(1, 2))

    gs = b // _BBS
    fw1t = f_w1.astype(bf16).T
    fw2t = f_w2.astype(bf16).T
    zs_tm, qp_p = pl.pallas_call(
        functools.partial(_scan_kernel, n_steps=n - 1),
        grid=(gs,),
        in_specs=[
            smem(), smem(), smem(),
            pl.BlockSpec((n, 1, _BBS), lambda i: (0, 0, i)),
            pl.BlockSpec((n, 1, _BBS), lambda i: (0, 0, i)),
            pl.BlockSpec((n - 1, 1, _BBS), lambda i: (0, 0, i)),
            pl.BlockSpec((1, _BBS), lambda i: (0, i)),
            full((hid, 3)), full((hid, 1)), full((1, hid)),
        ],
        out_specs=[
            pl.BlockSpec((n, 1, _BBS), lambda i: (0, 0, i)),
            smem((1, 1, 1), lambda i: (i, 0, 0)),
        ],
        out_shape=[
            jax.ShapeDtypeStruct((n, 1, b), f32),
            jax.ShapeDtypeStruct((gs, 1, 1), f32),
        ],
        compiler_params=pltpu.CompilerParams(
            dimension_semantics=("parallel",),
            vmem_limit_bytes=100 * 1024 * 1024,
        ),
    )(ts, sqdts, sc, c0_tm, c1_tm, bm_tm, z0_row,
      fw1t, f_b1.reshape(hid, 1), fw2t)

    gc = b // _BBC
    xh_tm, lp_p = pl.pallas_call(
        functools.partial(_proj_kernel, bb=_BBC),
        grid=(n, gc),
        in_specs=[
            smem(),
            pl.BlockSpec((1, 1, _BBC), lambda t, i: (t, 0, i)),
            pl.BlockSpec((1, 1, _BBC), lambda t, i: (t, 0, i)),
            full((1, hid)), full((1, hid)),
            full((hid, hid)), full((1, hid)),
            full((hid, 1)),
        ],
        out_specs=[
            pl.BlockSpec((1, 1, _BBC), lambda t, i: (t, 0, i)),
            smem((1, 1, 1), lambda t, i: (t * gc + i, 0, 0)),
        ],
        out_shape=[
            jax.ShapeDtypeStruct((n, 1, b), f32),
            jax.ShapeDtypeStruct((n * gc, 1, 1), f32),
        ],
        compiler_params=pltpu.CompilerParams(
            dimension_semantics=("parallel", "parallel"),
            vmem_limit_bytes=100 * 1024 * 1024,
        ),
    )(sc, zs_tm, xs_tm, proj_w1, proj_b1.reshape(1, hid),
      proj_w2, proj_b2.reshape(1, hid), proj_w3)

    inv_b = 1.0 / np.float32(b)
    log_pxs = jnp.sum(lp_p) * inv_b
    logqp = (jnp.sum(kl_p) + jnp.sum(qp_p)) * inv_b
    return log_pxs, logqp, xh_tm.reshape(n, b, 1)


# speedup vs baseline: 1.3019x; 1.0260x over previous
import functools

import jax
import jax.numpy as jnp
import numpy as np
from jax.experimental import pallas as pl
from jax.experimental.pallas import tpu as pltpu

f32 = jnp.float32
bf16 = jnp.bfloat16

_BBA = 2048
_BBS = 8192
_BBC = 2048


def _tlhs(a_t, w):
    return jax.lax.dot_general(a_t, w, (((0,), (0,)), ((), ())),
                               preferred_element_type=f32)


def _enc_kernel(sc_s, xs_r, eps_r, ew1_r, eb1_r, ew2_r, eb2_r, qzw_r, qzb_r,
                c0_r, c1_r, z0_r, okl_r, *, n):
    ew1c = jnp.swapaxes(ew1_r[...], 0, 1)
    eb1c = jnp.swapaxes(eb1_r[...], 0, 1)
    ew2 = ew2_r[...]
    eb2 = eb2_r[...]
    pzm = sc_s[0]
    pzls = sc_s[1]

    ctx0 = None
    for t in range(n):
        x = xs_r[t, 0:1, :]
        h1t = jnp.maximum(ew1c * x + eb1c, 0.0)
        ctx = _tlhs(h1t, ew2) + eb2
        ct_rows = jnp.swapaxes(ctx, 0, 1).astype(bf16)
        c0_r[t, 0:1, :] = ct_rows[0:1, :]
        c1_r[t, 0:1, :] = ct_rows[1:2, :]
        if t == 0:
            ctx0 = ctx.astype(bf16)

    q = jnp.dot(ctx0, qzw_r[...], preferred_element_type=f32) + qzb_r[...]
    q_rows = jnp.swapaxes(q, 0, 1)
    qm, qls = q_rows[0:1, :], q_rows[1:2, :]
    z0 = qm + jnp.exp(qls) * eps_r[...]
    kl = (pzls - qls) + (jnp.exp(2.0 * qls) + (qm - pzm) ** 2) / (
        2.0 * jnp.exp(2.0 * pzls)) - 0.5
    z0_r[...] = z0
    okl_r[0, 0, 0] = jnp.sum(kl)


def _scan_kernel(ts_s, sq_s, sc_s, c0_r, c1_r, bm_r, z0_r,
                 fw1t_r, fb1_r, fw2t_r, ozs_r, oqp_r, *, n_steps):
    kap = sc_s[2]
    th = sc_s[3]
    sig = sc_s[4]
    fb2 = sc_s[5]
    fw1t = fw1t_r[...]
    fb1 = fb1_r[...]
    fw2t = fw2t_r[...]

    z0 = z0_r[...]
    ozs_r[0, 0:1, :] = z0

    def body(t, carry):
        z, acc_q = carry
        zb = z.astype(bf16)
        c0 = c0_r[pl.ds(t + 1, 1), 0, :]
        c1 = c1_r[pl.ds(t + 1, 1), 0, :]
        in3 = jnp.concatenate([zb, c0, c1], axis=0)
        pre = jnp.dot(fw1t, in3, preferred_element_type=f32) + fb1
        h = jnp.maximum(pre, 0.0).astype(bf16)
        fr = jnp.dot(fw2t, h, preferred_element_type=f32) + fb2
        dt = ts_s[t + 1] - ts_s[t]
        u = (fr - kap * (th - z)) / sig
        acc_q = acc_q + (u * u) * (0.5 * dt)
        dw = bm_r[pl.ds(t, 1), 0, :] * sq_s[t]
        z = (z + fr * dt) + sig * dw
        ozs_r[pl.ds(t + 1, 1), 0, :] = z
        return z, acc_q

    _, acc_q = jax.lax.fori_loop(0, n_steps, body,
                                 (z0, jnp.zeros_like(z0)))
    oqp_r[0, 0, 0] = jnp.sum(acc_q)


def _proj_kernel(sc_s, zs_r, xs_r, pw1_r, pb1_r, pw2_r, pb2_r, pw3_r,
                 oxh_r, olp_r, *, bb):
    pb3 = sc_s[6]
    std = sc_s[7]
    pw1c = jnp.swapaxes(pw1_r[...], 0, 1)
    pb1c = jnp.swapaxes(pb1_r[...], 0, 1)

    z = zs_r[0, 0:1, :]
    a = pw1c * z + pb1c
    p1t = jnp.maximum(a, 0.0) + jnp.log1p(jnp.exp(-jnp.abs(a)))
    p2 = _tlhs(p1t, pw2_r[...]) + pb2_r[...]
    p2 = jnp.maximum(p2, 0.0).astype(bf16)
    xh_row = jax.lax.dot_general(pw3_r[...], p2, (((0,), (1,)), ((), ())),
                                 preferred_element_type=f32) + pb3
    oxh_r[0, 0:1, :] = xh_row
    r = (xs_r[0, 0:1, :] - xh_row) / std
    log2pi = np.float32(np.log(2.0 * np.pi))
    olp_r[0, 0, 0] = (-0.5 * jnp.sum(r * r)
                      - np.float32(bb) * (jnp.log(std) + 0.5 * log2pi))


def kernel(xs, ts, noise_std, eps0, bm_eps, enc_w1, enc_b1, enc_w2, enc_b2,
           qz0_w, qz0_b, f_w1, f_b1, f_w2, f_b2,
           proj_w1, proj_b1, proj_w2, proj_b2, proj_w3, proj_b3,
           pz0_mean, pz0_logstd, kappa, theta, sigma):
    n, b, _ = xs.shape
    hid = enc_w1.shape[1]

    sc = jnp.concatenate([
        pz0_mean.reshape(-1), pz0_logstd.reshape(-1),
        kappa.reshape(-1), theta.reshape(-1), sigma.reshape(-1),
        f_b2.reshape(-1), proj_b3.reshape(-1),
        jnp.asarray(noise_std, f32).reshape(-1),
    ]).astype(f32)
    sqdts = jnp.sqrt(ts[1:] - ts[:-1])

    xs_tm = xs.reshape(n, 1, b)
    bm_tm = bm_eps.reshape(n - 1, 1, b)
    eps_row = eps0.reshape(1, b)

    smem = functools.partial(pl.BlockSpec, memory_space=pltpu.SMEM)
    full = lambda shape: pl.BlockSpec(shape, lambda *i: (0,) * len(shape))

    ga = b // _BBA
    c0_tm, c1_tm, z0_row, kl_p = pl.pallas_call(
        functools.partial(_enc_kernel, n=n),
        grid=(ga,),
        in_specs=[
            smem(),
            pl.BlockSpec((n, 1, _BBA), lambda i: (0, 0, i)),
            pl.BlockSpec((1, _BBA), lambda i: (0, i)),
            full((1, hid)), full((1, hid)),
            full((hid, 2)), full((1, 2)),
            full((2, 2)), full((1, 2)),
        ],
        out_specs=[
            pl.BlockSpec((n, 1, _BBA), lambda i: (0, 0, i)),
            pl.BlockSpec((n, 1, _BBA), lambda i: (0, 0, i)),
            pl.BlockSpec((1, _BBA), lambda i: (0, i)),
            smem((1, 1, 1), lambda i: (i, 0, 0)),
        ],
        out_shape=[
            jax.ShapeDtypeStruct((n, 1, b), bf16),
            jax.ShapeDtypeStruct((n, 1, b), bf16),
            jax.ShapeDtypeStruct((1, b), f32),
            jax.ShapeDtypeStruct((ga, 1, 1), f32),
        ],
        compiler_params=pltpu.CompilerParams(
            dimension_semantics=("parallel",),
            vmem_limit_bytes=100 * 1024 * 1024,
        ),
    )(sc, xs_tm, eps_row, enc_w1, enc_b1.reshape(1, hid), enc_w2,
      enc_b2.reshape(1, 2), qz0_w, qz0_b.reshape(1, 2))

    gs = b // _BBS
    fw1t = f_w1.astype(bf16).T
    fw2t = f_w2.astype(bf16).T
    zs_tm, qp_p = pl.pallas_call(
        functools.partial(_scan_kernel, n_steps=n - 1),
        grid=(gs,),
        in_specs=[
            smem(), smem(), smem(),
            pl.BlockSpec((n, 1, _BBS), lambda i: (0, 0, i)),
            pl.BlockSpec((n, 1, _BBS), lambda i: (0, 0, i)),
            pl.BlockSpec((n - 1, 1, _BBS), lambda i: (0, 0, i)),
            pl.BlockSpec((1, _BBS), lambda i: (0, i)),
            full((hid, 3)), full((hid, 1)), full((1, hid)),
        ],
        out_specs=[
            pl.BlockSpec((n, 1, _BBS), lambda i: (0, 0, i)),
            smem((1, 1, 1), lambda i: (i, 0, 0)),
        ],
        out_shape=[
            jax.ShapeDtypeStruct((n, 1, b), f32),
            jax.ShapeDtypeStruct((gs, 1, 1), f32),
        ],
        compiler_params=pltpu.CompilerParams(
            dimension_semantics=("parallel",),
            vmem_limit_bytes=100 * 1024 * 1024,
        ),
    )(ts, sqdts, sc, c0_tm, c1_tm, bm_tm, z0_row,
      fw1t, f_b1.reshape(hid, 1), fw2t)

    gc = b // _BBC
    xh_tm, lp_p = pl.pallas_call(
        functools.partial(_proj_kernel, bb=_BBC),
        grid=(n, gc),
        in_specs=[
            smem(),
            pl.BlockSpec((1, 1, _BBC), lambda t, i: (t, 0, i)),
            pl.BlockSpec((1, 1, _BBC), lambda t, i: (t, 0, i)),
            full((1, hid)), full((1, hid)),
            full((hid, hid)), full((1, hid)),
            full((hid, 1)),
        ],
        out_specs=[
            pl.BlockSpec((1, 1, _BBC), lambda t, i: (t, 0, i)),
            smem((1, 1, 1), lambda t, i: (t * gc + i, 0, 0)),
        ],
        out_shape=[
            jax.ShapeDtypeStruct((n, 1, b), f32),
            jax.ShapeDtypeStruct((n * gc, 1, 1), f32),
        ],
        compiler_params=pltpu.CompilerParams(
            dimension_semantics=("parallel", "parallel"),
            vmem_limit_bytes=100 * 1024 * 1024,
        ),
    )(sc, zs_tm, xs_tm, proj_w1, proj_b1.reshape(1, hid),
      proj_w2, proj_b2.reshape(1, hid), proj_w3)

    inv_b = 1.0 / np.float32(b)
    log_pxs = jnp.sum(lp_p) * inv_b
    logqp = (jnp.sum(kl_p) + jnp.sum(qp_p)) * inv_b
    return log_pxs, logqp, xh_tm.reshape(n, b, 1)
